# baseline (speedup 1.0000x reference)
import functools

import jax
import jax.numpy as jnp
from jax import lax
from jax.experimental import pallas as pl
from jax.experimental.pallas import tpu as pltpu
from jax.experimental.pallas import tpu_sc as plsc

D = 32
L = 16
TW = 128
DH = D // 2


def _sorted_block_helpers(tab_h, blk, sem, ixv, lane, zero):
    def slots_of(g):
        vec = ixv[pl.ds(g * L, L)]
        cs = vec & -TW
        ph = vec & (TW - 1)
        slots = jnp.zeros((L,), jnp.int32)
        isnew = jnp.zeros((L,), jnp.int32)
        slot = zero
        nfetch = zero
        for j in range(L):
            if j == 0:
                is_new = jnp.bool_(True)
            else:
                is_new = cs[j] != cs[j - 1]
            slot = jnp.where(is_new, nfetch, slot)
            nfetch = nfetch + jnp.where(is_new, 1, 0)
            isnew = jnp.where(lane == j, jnp.where(is_new, 1, 0), isnew)
            slots = jnp.where(lane == j, slot, slots)
        return cs, ph, slots, isnew, nfetch

    def issue(cs, slots, isnew, h, cond):
        for j in range(L):
            off = pl.multiple_of(cs[j], TW)

            @pl.when((isnew[j] == 1) & cond)
            def _(off=off, j=j):
                pltpu.async_copy(
                    tab_h.at[pl.ds(h * DH, DH), pl.ds(off, TW)],
                    blk.at[slots[j]], sem)

    def wait(nfetch):
        def wait_one(k, carry):
            pltpu.make_async_copy(
                tab_h.at[pl.ds(0, DH), pl.ds(0, TW)], blk.at[0], sem).wait()
            return carry
        lax.fori_loop(0, nfetch, wait_one, 0)

    return slots_of, issue, wait


@functools.lru_cache(maxsize=None)
def _build_item(B):
    info = plsc.get_sparse_core_info()
    NC, NS = info.num_cores, info.num_subcores
    NW = NC * NS
    bpw = B // NW
    NG = bpw // L

    mesh = plsc.VectorSubcoreMesh(core_axis_name="c", subcore_axis_name="s")

    @functools.partial(
        pl.kernel,
        mesh=mesh,
        out_type=jax.ShapeDtypeStruct((B, TW), jnp.float32),
        compiler_params=pltpu.CompilerParams(
            needs_layout_passes=False, disable_bounds_checks=True),
        scratch_types=[
            pltpu.VMEM((bpw,), jnp.int32),
            pltpu.VMEM((NG, L), jnp.int32),
            pltpu.VMEM((L, DH, TW), jnp.float32),
            pltpu.VMEM((2, L, TW), jnp.float32),
            pltpu.SemaphoreType.DMA,
            pltpu.SemaphoreType.DMA,
            pltpu.SemaphoreType.DMA,
        ],
    )
    def item_kernel(sidx_h, rows_h, itabT_h, strip_h,
                    sixv, rowv, blk, stg, semf, sems0, sems1):
        cid = lax.axis_index("c")
        sid = lax.axis_index("s")
        wid = sid * NC + cid
        base = wid * bpw

        pltpu.sync_copy(sidx_h.at[pl.ds(base, bpw)], sixv)
        pltpu.sync_copy(rows_h.at[wid], rowv)
        lane = lax.iota(jnp.int32, L)
        zero = jnp.zeros((), jnp.int32)
        true_ = jnp.bool_(True)

        slots_of, issue, wait = _sorted_block_helpers(
            itabT_h, blk, semf, sixv, lane, zero)
        sems = (sems0, sems1)

        def scat_wait(par):
            pltpu.make_async_copy(
                strip_h.at[pl.ds(0, L)], stg.at[par], sems[par]).wait()

        def one_group(g, carry, par, k):
            cs, ph, slots, isnew, nfetch = carry
            rvec = rowv[g, :]
            for h in range(2):
                if h == 1:
                    issue(cs, slots, isnew, 1, true_)
                wait(nfetch)
                if h == 0:
                    @pl.when(k >= 1)
                    def _():
                        scat_wait(par)
                for d in range(DH):
                    dv = jnp.full((L,), d, dtype=jnp.int32)
                    vals = plsc.load_gather(blk, [slots, dv, ph])
                    col = (jnp.full((L,), h * DH + d, jnp.int32) + rvec) & (TW - 1)
                    plsc.store_scatter(stg.at[par], [lane, col], vals)

            gn = jnp.minimum(g + 1, NG - 1)
            nxt = slots_of(gn)
            issue(nxt[0], nxt[2], nxt[3], 0, g + 1 < NG)
            pltpu.async_copy(stg.at[par], strip_h.at[rowv.at[g]], sems[par])
            return nxt

        def pair(k, carry):
            carry = one_group(2 * k, carry, 0, k)
            carry = one_group(2 * k + 1, carry, 1, k)
            return carry

        first = slots_of(0)
        issue(first[0], first[2], first[3], 0, true_)
        lax.fori_loop(0, NG // 2, pair, first)
        scat_wait(0)
        scat_wait(1)

    return item_kernel


@functools.lru_cache(maxsize=None)
def _build_user(B):
    info = plsc.get_sparse_core_info()
    NC, NS = info.num_cores, info.num_subcores
    NW = NC * NS
    bpw = B // NW
    NG = bpw // L

    mesh = plsc.VectorSubcoreMesh(core_axis_name="c", subcore_axis_name="s")

    @functools.partial(
        pl.kernel,
        mesh=mesh,
        out_type=jax.ShapeDtypeStruct((NC, B), jnp.float32),
        compiler_params=pltpu.CompilerParams(
            needs_layout_passes=False, disable_bounds_checks=True),
        scratch_types=[
            pltpu.VMEM((bpw,), jnp.int32),
            pltpu.VMEM((NG, L), jnp.int32),
            pltpu.VMEM((L, DH, TW), jnp.float32),
            pltpu.VMEM((2, L, TW), jnp.float32),
            pltpu.VMEM((D, L), jnp.float32),
            pltpu.VMEM((D,), jnp.float32),
            pltpu.VMEM((L,), jnp.float32),
            pltpu.VMEM((L,), jnp.float32),
            pltpu.VMEM((B // NS,), jnp.float32),
            pltpu.VMEM_SHARED((B,), jnp.float32),
            pltpu.SemaphoreType.DMA,
            pltpu.SemaphoreType.DMA,
            pltpu.SemaphoreType.DMA,
        ],
    )
    def user_kernel(sidx_h, rows_h, utabT_h, strip_h, w_h, b_h, out_h,
                    uixv, rowv, blk, sbuf, stage, wv, bv, accv, winv,
                    slab, semf, semg0, semg1):
        cid = lax.axis_index("c")
        sid = lax.axis_index("s")
        wid = sid * NC + cid
        base = wid * bpw
        win = B // NS

        pltpu.sync_copy(sidx_h.at[pl.ds(base, bpw)], uixv)
        pltpu.sync_copy(rows_h.at[wid], rowv)
        pltpu.sync_copy(w_h, wv)
        pltpu.sync_copy(b_h, bv)

        def zfill(k, carry):
            winv[pl.ds(k * L, L)] = jnp.zeros((L,), jnp.float32)
            return carry
        lax.fori_loop(0, win // L, zfill, 0)
        pltpu.sync_copy(winv, slab.at[pl.ds(sid * win, win)])
        plsc.subcore_barrier()

        w_lo = wv[pl.ds(0, L)]
        w_hi = wv[pl.ds(L, L)]
        bvec = bv[...]
        lane = lax.iota(jnp.int32, L)
        zero = jnp.zeros((), jnp.int32)
        true_ = jnp.bool_(True)

        slots_of, issue, wait = _sorted_block_helpers(
            utabT_h, blk, semf, uixv, lane, zero)
        semg = (semg0, semg1)

        def strip_issue(g, par):
            pltpu.async_copy(
                strip_h.at[rowv.at[g]], sbuf.at[par], semg[par])

        def strip_wait(par):
            pltpu.make_async_copy(
                strip_h.at[pl.ds(0, L)], sbuf.at[par], semg[par]).wait()

        def one_group(g, carry, par):
            cs, ph, slots, isnew, nfetch = carry
            for h in range(2):
                if h == 1:
                    issue(cs, slots, isnew, 1, true_)
                wait(nfetch)
                for d in range(DH):
                    dv = jnp.full((L,), d, dtype=jnp.int32)
                    w_d = w_lo[d] if h == 0 else w_hi[d]
                    stage[h * DH + d, :] = (
                        plsc.load_gather(blk, [slots, dv, ph]) * w_d)

            gn = jnp.minimum(g + 1, NG - 1)
            nxt = slots_of(gn)
            issue(nxt[0], nxt[2], nxt[3], 0, g + 1 < NG)

            @pl.when(g + 1 < NG)
            def _():
                strip_issue(gn, 1 - par)

            strip_wait(par)
            rvec = rowv[g, :]
            acc = bvec
            for d in range(D):
                dv = jnp.full((L,), d, dtype=jnp.int32)
                col = (dv + rvec) & (TW - 1)
                acc = acc + stage[d, :] * plsc.load_gather(
                    sbuf.at[par], [lane, col])
            accv[...] = acc
            pltpu.sync_copy(accv, slab.at[rowv.at[g]])
            return nxt

        def pair(k, carry):
            carry = one_group(2 * k, carry, 0)
            carry = one_group(2 * k + 1, carry, 1)
            return carry

        first = slots_of(0)
        issue(first[0], first[2], first[3], 0, true_)
        strip_issue(0, 0)
        lax.fori_loop(0, NG // 2, pair, first)

        plsc.subcore_barrier()
        pltpu.sync_copy(slab.at[pl.ds(sid * win, win)],
                        out_h.at[cid, pl.ds(sid * win, win)])

    return user_kernel


def kernel(user_idx, item_idx, user_table, item_table, W, b):
    B = user_idx.shape[0]
    info = plsc.get_sparse_core_info()
    NW = info.num_cores * info.num_subcores
    ng = (B // NW) // L
    rows = lax.iota(jnp.int32, B)
    si, pi = lax.sort_key_val(item_idx, rows)
    strip = _build_item(B)(si, pi.reshape(NW, ng, L), item_table.T)
    su, pu = lax.sort_key_val(user_idx, rows)
    out2 = _build_user(B)(
        su, pu.reshape(NW, ng, L), user_table.T, strip,
        W.reshape(-1), jnp.broadcast_to(b, (L,)))
    return out2.sum(axis=0)


# speedup vs baseline: 1.1348x; 1.1348x over previous
import functools

import jax
import jax.numpy as jnp
from jax import lax
from jax.experimental import pallas as pl
from jax.experimental.pallas import tpu as pltpu
from jax.experimental.pallas import tpu_sc as plsc

D = 32
L = 16
TW = 128
DH = D // 2


@functools.lru_cache(maxsize=None)
def _build(B):
    info = plsc.get_sparse_core_info()
    NC, NS = info.num_cores, info.num_subcores
    NW = NC * NS
    bpw = B // NW
    NG = bpw // L

    mesh = plsc.VectorSubcoreMesh(core_axis_name="c", subcore_axis_name="s")

    @functools.partial(
        pl.kernel,
        mesh=mesh,
        out_type=jax.ShapeDtypeStruct((NC, B), jnp.float32),
        compiler_params=pltpu.CompilerParams(
            needs_layout_passes=False, disable_bounds_checks=True),
        scratch_types=[
            pltpu.VMEM((bpw,), jnp.int32),
            pltpu.VMEM((bpw,), jnp.int32),
            pltpu.VMEM((NG, L), jnp.int32),
            pltpu.VMEM((L, DH, TW), jnp.float32),
            pltpu.VMEM((L, D, TW), jnp.float32),
            pltpu.VMEM((D, L), jnp.float32),
            pltpu.VMEM((D,), jnp.float32),
            pltpu.VMEM((L,), jnp.float32),
            pltpu.VMEM((L,), jnp.float32),
            pltpu.VMEM((B // NS,), jnp.float32),
            pltpu.VMEM_SHARED((B,), jnp.float32),
            pltpu.SemaphoreType.DMA,
            pltpu.SemaphoreType.DMA,
        ],
    )
    def sc_kernel(uidx_h, iidx_h, perm_h, utabT_h, itabT_h, w_h, b_h, out_h,
                  uixv, iixv, rowv, ublk, iblk, stage, wv, bv, accv, winv,
                  slab, semu, semi):
        cid = lax.axis_index("c")
        sid = lax.axis_index("s")
        wid = sid * NC + cid
        base = wid * bpw
        win = B // NS

        pltpu.sync_copy(uidx_h.at[pl.ds(base, bpw)], uixv)
        pltpu.sync_copy(iidx_h.at[pl.ds(base, bpw)], iixv)
        pltpu.sync_copy(perm_h.at[wid], rowv)
        pltpu.sync_copy(w_h, wv)
        pltpu.sync_copy(b_h, bv)

        def zfill(k, carry):
            winv[pl.ds(k * L, L)] = jnp.zeros((L,), jnp.float32)
            return carry
        lax.fori_loop(0, win // L, zfill, 0)
        pltpu.sync_copy(winv, slab.at[pl.ds(sid * win, win)])
        plsc.subcore_barrier()

        w_lo = wv[pl.ds(0, L)]
        w_hi = wv[pl.ds(L, L)]
        bvec = bv[...]
        lane = lax.iota(jnp.int32, L)
        zero = jnp.zeros((), jnp.int32)

        def user_slots(g):
            uvec = uixv[pl.ds(g * L, L)]
            cs = uvec & -TW
            uph = uvec & (TW - 1)
            slots = jnp.zeros((L,), jnp.int32)
            isnew = jnp.zeros((L,), jnp.int32)
            slot = zero
            nfetch = zero
            for j in range(L):
                if j == 0:
                    is_new = jnp.bool_(True)
                else:
                    is_new = cs[j] != cs[j - 1]
                slot = jnp.where(is_new, nfetch, slot)
                nfetch = nfetch + jnp.where(is_new, 1, 0)
                isnew = jnp.where(lane == j, jnp.where(is_new, 1, 0), isnew)
                slots = jnp.where(lane == j, slot, slots)
            return cs, uph, slots, isnew, nfetch

        def issue_user(cs, slots, isnew, h, cond):
            for j in range(L):
                off = pl.multiple_of(cs[j], TW)

                @pl.when((isnew[j] == 1) & cond)
                def _(off=off, j=j):
                    pltpu.async_copy(
                        utabT_h.at[pl.ds(h * DH, DH), pl.ds(off, TW)],
                        ublk.at[slots[j]], semu)

        def wait_user(nfetch):
            def wait_one(k, carry):
                pltpu.make_async_copy(
                    utabT_h.at[pl.ds(0, DH), pl.ds(0, TW)],
                    ublk.at[0], semu).wait()
                return carry
            lax.fori_loop(0, nfetch, wait_one, 0)

        true_ = jnp.bool_(True)

        def group(g, carry):
            ucs, uph, slots, isnew, nfetch = carry

            ivec = iixv[pl.ds(g * L, L)]
            ics = ivec & -TW
            for j in range(L):
                off = pl.multiple_of(ics[j], TW)
                pltpu.async_copy(
                    itabT_h.at[:, pl.ds(off, TW)], iblk.at[j], semi)

            for h in range(2):
                if h == 1:
                    issue_user(ucs, slots, isnew, 1, true_)
                wait_user(nfetch)
                for d in range(DH):
                    dv = jnp.full((L,), d, dtype=jnp.int32)
                    w_d = w_lo[d] if h == 0 else w_hi[d]
                    stage[h * DH + d, :] = (
                        plsc.load_gather(ublk, [slots, dv, uph]) * w_d)

            gn = jnp.minimum(g + 1, NG - 1)
            nxt = user_slots(gn)
            issue_user(nxt[0], nxt[2], nxt[3], 0, g + 1 < NG)

            for j in range(L):
                pltpu.make_async_copy(
                    itabT_h.at[:, pl.ds(0, TW)], iblk.at[j], semi).wait()

            iph = ivec & (TW - 1)
            acc = bvec
            for d in range(D):
                dv = jnp.full((L,), d, dtype=jnp.int32)
                acc = acc + stage[d, :] * plsc.load_gather(iblk, [lane, dv, iph])
            accv[...] = acc
            pltpu.sync_copy(accv, slab.at[rowv.at[g]])
            return nxt

        first = user_slots(0)
        issue_user(first[0], first[2], first[3], 0, true_)
        lax.fori_loop(0, NG, group, first)

        plsc.subcore_barrier()
        pltpu.sync_copy(slab.at[pl.ds(sid * win, win)],
                        out_h.at[cid, pl.ds(sid * win, win)])

    return sc_kernel


def kernel(user_idx, item_idx, user_table, item_table, W, b):
    B = user_idx.shape[0]
    info = plsc.get_sparse_core_info()
    NW = info.num_cores * info.num_subcores
    rows = lax.iota(jnp.int32, B)
    su, perm = lax.sort_key_val(user_idx, rows)
    si = jnp.take(item_idx, perm)
    out2 = _build(B)(
        su, si, perm.reshape(NW, (B // NW) // L, L), user_table.T,
        item_table.T, W.reshape(-1), jnp.broadcast_to(b, (L,)))
    return out2.sum(axis=0)
